# Initial kernel scaffold; baseline (speedup 1.0000x reference)
import jax
import jax.numpy as jnp
from jax.experimental import pallas as pl


def kernel(x, edge_index, W, att_src, att_dst, bias, mlp_W, mlp_b):
    raise NotImplementedError("write your pallas kernel here")


# speedup vs baseline: 32.9340x; 32.9340x over previous
import functools
import jax
import jax.numpy as jnp
from jax import lax
from jax.experimental import pallas as pl
from jax.experimental.pallas import tpu as pltpu
from jax.experimental.pallas import tpu_sc as plsc

_N = 10000
_E = 320000
_D = 128
_H = 8
_C = 8
_HC = _H * _C
_NC = 2
_NS = 16
_NW = _NC * _NS
_ET = _E // _NW
_CH = 80
_ITERS = _ET // _CH
_STRIPE = 632
_NP = _STRIPE * _NS
_BLK = 400
_F32 = jnp.float32


def _elu(v):
    return jnp.where(v > 0, v, jnp.exp(v) - 1.0)


def _tc_pre_body(x_ref, w_ref, a_ref, xs_ref, t1_ref, t2_ref):
    xs = jnp.dot(x_ref[...], w_ref[...], preferred_element_type=_F32)
    xs_ref[...] = xs
    t12 = jnp.dot(xs, a_ref[...], preferred_element_type=_F32)
    t1_ref[...] = t12[:, :16]
    t2_ref[...] = t12[:, 16:]


def _tc_pre(x, W, A):
    grid = _N // _BLK
    return pl.pallas_call(
        _tc_pre_body,
        grid=(grid,),
        in_specs=[
            pl.BlockSpec((_BLK, _D), lambda i: (i, 0)),
            pl.BlockSpec((_D, _HC), lambda i: (0, 0)),
            pl.BlockSpec((_HC, 32), lambda i: (0, 0)),
        ],
        out_specs=[
            pl.BlockSpec((_BLK, _HC), lambda i: (i, 0)),
            pl.BlockSpec((_BLK, 16), lambda i: (i, 0)),
            pl.BlockSpec((_BLK, 16), lambda i: (i, 0)),
        ],
        out_shape=[
            jax.ShapeDtypeStruct((_N, _HC), _F32),
            jax.ShapeDtypeStruct((_N, 16), _F32),
            jax.ShapeDtypeStruct((_N, 16), _F32),
        ],
    )(x, W, A)


def _tc_post_body(a0_ref, a1_ref, d0_ref, d1_ref, b_ref, mw_ref, mb_ref,
                  r_ref, m_ref, out_ref, dent_ref):
    den = d0_ref[...] + d1_ref[...]
    dent_ref[...] = den
    agg = a0_ref[...] + a1_ref[...]
    den64 = jnp.dot(den, r_ref[...], preferred_element_type=_F32) + 1e-16
    agn = agg / den64
    m = jnp.dot(agn, m_ref[...], preferred_element_type=_F32) + b_ref[...]
    m = _elu(m)
    h2 = jnp.dot(m, mw_ref[...], preferred_element_type=_F32) + mb_ref[...]
    out_ref[...] = _elu(h2)


def _tc_post(a0, a1, d0, d1, bias2, mlp_W, mlp_b2, R16, M):
    grid = _N // _BLK
    return pl.pallas_call(
        _tc_post_body,
        grid=(grid,),
        in_specs=[
            pl.BlockSpec((_BLK, _HC), lambda i: (i, 0)),
            pl.BlockSpec((_BLK, _HC), lambda i: (i, 0)),
            pl.BlockSpec((_BLK, 16), lambda i: (i, 0)),
            pl.BlockSpec((_BLK, 16), lambda i: (i, 0)),
            pl.BlockSpec((1, _C), lambda i: (0, 0)),
            pl.BlockSpec((_C, 16), lambda i: (0, 0)),
            pl.BlockSpec((1, 16), lambda i: (0, 0)),
            pl.BlockSpec((16, _HC), lambda i: (0, 0)),
            pl.BlockSpec((_HC, _C), lambda i: (0, 0)),
        ],
        out_specs=[
            pl.BlockSpec((_BLK, 16), lambda i: (i, 0)),
            pl.BlockSpec((_BLK, 16), lambda i: (i, 0)),
        ],
        out_shape=[
            jax.ShapeDtypeStruct((_N, 16), _F32),
            jax.ShapeDtypeStruct((_N, 16), _F32),
        ],
    )(a0, a1, d0, d1, bias2, mlp_W, mlp_b2, R16, M)


def _sc1_body(src_h, dst_h, t1_h, t2_h, xs_h, ex_h, denp_h, aggp_h,
              idx_s, idx_d, buf_s, buf_d, exbuf, xsbuf, msgbuf,
              den_sh, agg_sh, sem1, sem2, sem3):
    c = lax.axis_index("c")
    s = lax.axis_index("s")
    wid = c * _NS + s
    rbase = pl.multiple_of(s * _STRIPE, 8)

    lane = lax.iota(jnp.int32, 16)
    pks = [2 * k + (lane >> 3) for k in range(4)]
    zero16 = jnp.zeros((16,), _F32)

    def zbody(j, carry):
        buf_s[j, :] = zero16
        for k in range(4):
            msgbuf[j, pl.ds(16 * k, 16)] = zero16
        return carry

    lax.fori_loop(0, _CH, zbody, 0)
    for off in range(0, _STRIPE, _CH):
        n = min(_CH, _STRIPE - off)
        ro = pl.multiple_of(rbase + off, 8)
        pltpu.sync_copy(msgbuf.at[pl.ds(0, n)], agg_sh.at[pl.ds(ro, n)])
        pltpu.sync_copy(buf_s.at[pl.ds(0, n)], den_sh.at[pl.ds(ro, n)])
    plsc.subcore_barrier()

    def ebody(i, carry):
        eb = pl.multiple_of(wid * _ET + i * _CH, 8)
        pltpu.sync_copy(src_h.at[pl.ds(eb, _CH)], idx_s)
        pltpu.sync_copy(dst_h.at[pl.ds(eb, _CH)], idx_d)
        cp1 = pltpu.async_copy(t1_h.at[idx_s], buf_s, sem1)
        cp2 = pltpu.async_copy(t2_h.at[idx_d], buf_d, sem2)
        cp3 = pltpu.async_copy(xs_h.at[idx_s], xsbuf, sem3)
        cp1.wait()
        cp2.wait()
        cp3.wait()

        def jbody(j, jcarry):
            al = buf_s[j, :] + buf_d[j, :]
            al = jnp.where(al > 0, al, al * 0.2)
            ev = jnp.exp(al)
            exbuf[j, :] = ev
            jv = jnp.full((16,), j, jnp.int32)
            for k in range(4):
                xq = xsbuf[j, pl.ds(16 * k, 16)]
                mlt = plsc.load_gather(exbuf, [jv, pks[k]])
                msgbuf[j, pl.ds(16 * k, 16)] = xq * mlt
            return jcarry

        lax.fori_loop(0, _CH, jbody, 0, unroll=2)
        pltpu.sync_copy(exbuf, ex_h.at[pl.ds(eb, _CH)])
        pltpu.sync_copy(exbuf, den_sh.at[idx_d], add=True)
        pltpu.sync_copy(msgbuf, agg_sh.at[idx_d], add=True)
        return carry

    lax.fori_loop(0, _ITERS, ebody, 0)
    plsc.subcore_barrier()

    for off in range(0, _STRIPE, _CH):
        n = min(_CH, _STRIPE - off)
        ro = pl.multiple_of(rbase + off, 8)
        pltpu.sync_copy(den_sh.at[pl.ds(ro, n)],
                        denp_h.at[c].at[pl.ds(ro, n)])
        pltpu.sync_copy(agg_sh.at[pl.ds(ro, n)],
                        aggp_h.at[c].at[pl.ds(ro, n)])


def _sc1(src, dst, t1_tab, t2_tab, xs_tab):
    mesh = plsc.VectorSubcoreMesh(core_axis_name="c", subcore_axis_name="s", num_cores=_NC, num_subcores=_NS)
    f = pl.kernel(
        _sc1_body,
        out_type=[
            jax.ShapeDtypeStruct((_E, 16), _F32),
            jax.ShapeDtypeStruct((_NC, _NP, 16), _F32),
            jax.ShapeDtypeStruct((_NC, _NP, _HC), _F32),
        ],
        mesh=mesh,
        scratch_types=[
            pltpu.VMEM((_CH,), jnp.int32),
            pltpu.VMEM((_CH,), jnp.int32),
            pltpu.VMEM((_CH, 16), _F32),
            pltpu.VMEM((_CH, 16), _F32),
            pltpu.VMEM((_CH, 16), _F32),
            pltpu.VMEM((_CH, _HC), _F32),
            pltpu.VMEM((_CH, _HC), _F32),
            pltpu.VMEM_SHARED((_NP, 16), _F32),
            pltpu.VMEM_SHARED((_NP, _HC), _F32),
            pltpu.SemaphoreType.DMA,
            pltpu.SemaphoreType.DMA,
            pltpu.SemaphoreType.DMA,
        ],
        compiler_params=pltpu.CompilerParams(needs_layout_passes=False, use_tc_tiling_on_sc=False),
    )
    return f(src, dst, t1_tab, t2_tab, xs_tab)


def _sc2_body(dst_h, ex_h, den_h, at_h, idx_d, exb, denb, atb, sem1):
    c = lax.axis_index("c")
    s = lax.axis_index("s")
    wid = c * _NS + s

    def ebody(i, carry):
        eb = wid * _ET + i * _CH
        pltpu.sync_copy(dst_h.at[pl.ds(eb, _CH)], idx_d)
        pltpu.sync_copy(ex_h.at[pl.ds(eb, _CH)], exb)
        pltpu.async_copy(den_h.at[idx_d], denb, sem1).wait()

        def jbody(j, jcarry):
            atb[j, :] = exb[j, :] / (denb[j, :] + 1e-16)
            return jcarry

        lax.fori_loop(0, _CH, jbody, 0, unroll=4)
        pltpu.sync_copy(atb, at_h.at[pl.ds(eb, _CH)])
        return carry

    lax.fori_loop(0, _ITERS, ebody, 0)


def _sc2(dst, ex_h, dent):
    mesh = plsc.VectorSubcoreMesh(core_axis_name="c", subcore_axis_name="s", num_cores=_NC, num_subcores=_NS)
    f = pl.kernel(
        _sc2_body,
        out_type=[jax.ShapeDtypeStruct((_E, 16), _F32)],
        mesh=mesh,
        scratch_types=[
            pltpu.VMEM((_CH,), jnp.int32),
            pltpu.VMEM((_CH, 16), _F32),
            pltpu.VMEM((_CH, 16), _F32),
            pltpu.VMEM((_CH, 16), _F32),
            pltpu.SemaphoreType.DMA,
        ],
        compiler_params=pltpu.CompilerParams(needs_layout_passes=False, use_tc_tiling_on_sc=False),
    )
    return f(dst, ex_h, dent)[0]


def kernel(x, edge_index, W, att_src, att_dst, bias, mlp_W, mlp_b):
    ei = edge_index.T
    src = ei[0].astype(jnp.int32)
    dst = ei[1].astype(jnp.int32)

    eye8 = jnp.eye(_C, dtype=_F32)
    a_src = (att_src[:, :, None] * eye8[:, None, :]).reshape(_HC, _H)
    a_dst = (att_dst[:, :, None] * eye8[:, None, :]).reshape(_HC, _H)
    A = jnp.concatenate([a_src, a_dst, a_dst, a_src], axis=1)
    R16 = jnp.concatenate(
        [jnp.repeat(eye8, _C, axis=1), jnp.zeros((_C, _HC), _F32)], axis=0)
    M = jnp.tile(eye8, (_H, 1)) / _H

    xs_tab, t1_tab, t2_tab = _tc_pre(x, W, A)
    ex_h, denp, aggp = _sc1(src, dst, t1_tab, t2_tab, xs_tab)
    out, dent = _tc_post(aggp[0, :_N], aggp[1, :_N], denp[0, :_N], denp[1, :_N],
                         bias.reshape(1, _C), mlp_W, mlp_b.reshape(1, 16),
                         R16, M)
    attn16 = _sc2(dst, ex_h, dent)
    attn = attn16[:, :_H]
    return out, (ei, attn)


# speedup vs baseline: 47.8596x; 1.4532x over previous
import functools
import jax
import jax.numpy as jnp
from jax import lax
from jax.experimental import pallas as pl
from jax.experimental.pallas import tpu as pltpu
from jax.experimental.pallas import tpu_sc as plsc

_N = 10000
_E = 320000
_D = 128
_H = 8
_C = 8
_HC = _H * _C
_NC = 2
_NS = 16
_NW = _NC * _NS
_ET = _E // _NW
_CH = 80
_ITERS = _ET // _CH
_STRIPE = 632
_NP = _STRIPE * _NS
_BLK = 400
_F32 = jnp.float32


def _elu(v):
    return jnp.where(v > 0, v, jnp.exp(v) - 1.0)


def _tc_pre_body(x_ref, w_ref, a_ref, xs_ref, t1_ref, t2_ref):
    xs = jnp.dot(x_ref[...], w_ref[...], preferred_element_type=_F32)
    xs_ref[...] = xs
    t12 = jnp.dot(xs, a_ref[...], preferred_element_type=_F32)
    t1_ref[...] = t12[:, :16]
    t2_ref[...] = t12[:, 16:]


def _tc_pre(x, W, A):
    grid = _N // _BLK
    return pl.pallas_call(
        _tc_pre_body,
        grid=(grid,),
        in_specs=[
            pl.BlockSpec((_BLK, _D), lambda i: (i, 0)),
            pl.BlockSpec((_D, _HC), lambda i: (0, 0)),
            pl.BlockSpec((_HC, 32), lambda i: (0, 0)),
        ],
        out_specs=[
            pl.BlockSpec((_BLK, _HC), lambda i: (i, 0)),
            pl.BlockSpec((_BLK, 16), lambda i: (i, 0)),
            pl.BlockSpec((_BLK, 16), lambda i: (i, 0)),
        ],
        out_shape=[
            jax.ShapeDtypeStruct((_N, _HC), _F32),
            jax.ShapeDtypeStruct((_N, 16), _F32),
            jax.ShapeDtypeStruct((_N, 16), _F32),
        ],
    )(x, W, A)


def _tc_post_body(ap_ref, dp_ref, b_ref, mw_ref, mb_ref,
                  r_ref, m_ref, out_ref, dent_ref):
    den = dp_ref[0] + dp_ref[1]
    dent_ref[...] = den
    agg = ap_ref[0] + ap_ref[1]
    den64 = jnp.dot(den, r_ref[...], preferred_element_type=_F32) + 1e-16
    agn = agg / den64
    m = jnp.dot(agn, m_ref[...], preferred_element_type=_F32) + b_ref[...]
    m = _elu(m)
    h2 = jnp.dot(m, mw_ref[...], preferred_element_type=_F32) + mb_ref[...]
    out_ref[...] = _elu(h2)


def _tc_post(ap, dp, bias2, mlp_W, mlp_b2, R16, M):
    grid = _N // _BLK
    return pl.pallas_call(
        _tc_post_body,
        grid=(grid,),
        in_specs=[
            pl.BlockSpec((2, _BLK, _HC), lambda i: (0, i, 0)),
            pl.BlockSpec((2, _BLK, 16), lambda i: (0, i, 0)),
            pl.BlockSpec((1, _C), lambda i: (0, 0)),
            pl.BlockSpec((_C, 16), lambda i: (0, 0)),
            pl.BlockSpec((1, 16), lambda i: (0, 0)),
            pl.BlockSpec((16, _HC), lambda i: (0, 0)),
            pl.BlockSpec((_HC, _C), lambda i: (0, 0)),
        ],
        out_specs=[
            pl.BlockSpec((_BLK, 16), lambda i: (i, 0)),
            pl.BlockSpec((_BLK, 16), lambda i: (i, 0)),
        ],
        out_shape=[
            jax.ShapeDtypeStruct((_N, 16), _F32),
            jax.ShapeDtypeStruct((_N, 16), _F32),
        ],
    )(ap, dp, bias2, mlp_W, mlp_b2, R16, M)


def _sc1_body(src3_h, dst3_h, t1_h, t2_h, xs_h, ex_h, denp_h, aggp_h,
              isrc, idst, buf_s0, buf_d0, xsb0, buf_s1, buf_d1, xsb1,
              exb0, msgb0, exb1, msgb1, den_sh, agg_sh,
              sa0, sb0, sx0, sa1, sb1, sx1):
    c = lax.axis_index("c")
    s = lax.axis_index("s")
    wid = c * _NS + s
    rbase = pl.multiple_of(s * _STRIPE, 8)
    ebase = wid * _ET

    lane = lax.iota(jnp.int32, 16)
    pks = [2 * k + (lane >> 3) for k in range(4)]
    zero16 = jnp.zeros((16,), _F32)

    slots = (
        (buf_s0, buf_d0, xsb0, exb0, msgb0, sa0, sb0, sx0),
        (buf_s1, buf_d1, xsb1, exb1, msgb1, sa1, sb1, sx1),
    )

    def zbody(j, carry):
        buf_s0[j, :] = zero16
        for k in range(4):
            msgb0[j, pl.ds(16 * k, 16)] = zero16
        return carry

    lax.fori_loop(0, _CH, zbody, 0)
    for off in range(0, _STRIPE, _CH):
        n = min(_CH, _STRIPE - off)
        ro = pl.multiple_of(rbase + off, 8)
        pltpu.sync_copy(msgb0.at[pl.ds(0, n)], agg_sh.at[pl.ds(ro, n)])
        pltpu.sync_copy(buf_s0.at[pl.ds(0, n)], den_sh.at[pl.ds(ro, n)])
    plsc.subcore_barrier()

    pltpu.sync_copy(src3_h.at[wid], isrc)
    pltpu.sync_copy(dst3_h.at[wid], idst)

    def fire(i, sl):
        b_s, b_d, b_x, _, _, s1, s2, s3 = sl
        pltpu.async_copy(t1_h.at[isrc.at[i]], b_s, s1)
        pltpu.async_copy(t2_h.at[idst.at[i]], b_d, s2)
        pltpu.async_copy(xs_h.at[isrc.at[i]], b_x, s3)

    def drain(i, sl):
        b_s, b_d, b_x, _, _, s1, s2, s3 = sl
        pltpu.make_async_copy(t1_h.at[isrc.at[i]], b_s, s1).wait()
        pltpu.make_async_copy(t2_h.at[idst.at[i]], b_d, s2).wait()
        pltpu.make_async_copy(xs_h.at[isrc.at[i]], b_x, s3).wait()

    def compute(i, sl):
        b_s, b_d, b_x, exb, msgb, _, _, _ = sl

        def jbody(j, jcarry):
            al = b_s[j, :] + b_d[j, :]
            al = jnp.where(al > 0, al, al * 0.2)
            ev = jnp.exp(al)
            exb[j, :] = ev
            jv = jnp.full((16,), j, jnp.int32)
            for k in range(4):
                xq = b_x[j, pl.ds(16 * k, 16)]
                mlt = plsc.load_gather(exb, [jv, pks[k]])
                msgb[j, pl.ds(16 * k, 16)] = xq * mlt
            return jcarry

        lax.fori_loop(0, _CH, jbody, 0, unroll=4)
        eb = pl.multiple_of(ebase + i * _CH, 8)
        pltpu.sync_copy(exb, ex_h.at[pl.ds(eb, _CH)])
        pltpu.sync_copy(exb, den_sh.at[idst.at[i]], add=True)
        pltpu.sync_copy(msgb, agg_sh.at[idst.at[i]], add=True)

    fire(0, slots[0])

    def gbody(g, carry):
        i0 = 2 * g
        drain(i0, slots[0])
        fire(i0 + 1, slots[1])
        compute(i0, slots[0])
        drain(i0 + 1, slots[1])
        fire(i0 + 2, slots[0])
        compute(i0 + 1, slots[1])
        return carry

    lax.fori_loop(0, (_ITERS - 1) // 2, gbody, 0)
    drain(_ITERS - 1, slots[0])
    compute(_ITERS - 1, slots[0])
    plsc.subcore_barrier()

    for off in range(0, _STRIPE, _CH):
        n = min(_CH, _STRIPE - off)
        ro = pl.multiple_of(rbase + off, 8)
        pltpu.sync_copy(den_sh.at[pl.ds(ro, n)],
                        denp_h.at[c].at[pl.ds(ro, n)])
        pltpu.sync_copy(agg_sh.at[pl.ds(ro, n)],
                        aggp_h.at[c].at[pl.ds(ro, n)])


def _sc1(src3, dst3, t1_tab, t2_tab, xs_tab):
    mesh = plsc.VectorSubcoreMesh(core_axis_name="c", subcore_axis_name="s", num_cores=_NC, num_subcores=_NS)
    f = pl.kernel(
        _sc1_body,
        out_type=[
            jax.ShapeDtypeStruct((_E, 16), _F32),
            jax.ShapeDtypeStruct((_NC, _NP, 16), _F32),
            jax.ShapeDtypeStruct((_NC, _NP, _HC), _F32),
        ],
        mesh=mesh,
        scratch_types=[
            pltpu.VMEM((_ITERS, _CH), jnp.int32),
            pltpu.VMEM((_ITERS, _CH), jnp.int32),
            pltpu.VMEM((_CH, 16), _F32),
            pltpu.VMEM((_CH, 16), _F32),
            pltpu.VMEM((_CH, _HC), _F32),
            pltpu.VMEM((_CH, 16), _F32),
            pltpu.VMEM((_CH, 16), _F32),
            pltpu.VMEM((_CH, _HC), _F32),
            pltpu.VMEM((_CH, 16), _F32),
            pltpu.VMEM((_CH, _HC), _F32),
            pltpu.VMEM((_CH, 16), _F32),
            pltpu.VMEM((_CH, _HC), _F32),
            pltpu.VMEM_SHARED((_NP, 16), _F32),
            pltpu.VMEM_SHARED((_NP, _HC), _F32),
            pltpu.SemaphoreType.DMA,
            pltpu.SemaphoreType.DMA,
            pltpu.SemaphoreType.DMA,
            pltpu.SemaphoreType.DMA,
            pltpu.SemaphoreType.DMA,
            pltpu.SemaphoreType.DMA,
        ],
        compiler_params=pltpu.CompilerParams(needs_layout_passes=False, use_tc_tiling_on_sc=False),
    )
    return f(src3, dst3, t1_tab, t2_tab, xs_tab)


def _sc2_body(dst3_h, ex_h, den_h, at_h, idst,
              exb0, dnb0, atb0, exb1, dnb1, atb1, se0, sd0, se1, sd1):
    c = lax.axis_index("c")
    s = lax.axis_index("s")
    wid = c * _NS + s
    ebase = wid * _ET

    pltpu.sync_copy(dst3_h.at[wid], idst)

    slots = ((exb0, dnb0, atb0, se0, sd0), (exb1, dnb1, atb1, se1, sd1))

    def fire(i, sl):
        exb, dnb, _, s1, s2 = sl
        eb = pl.multiple_of(ebase + i * _CH, 8)
        pltpu.async_copy(ex_h.at[pl.ds(eb, _CH)], exb, s1)
        pltpu.async_copy(den_h.at[idst.at[i]], dnb, s2)

    def drain(i, sl):
        exb, dnb, _, s1, s2 = sl
        eb = pl.multiple_of(ebase + i * _CH, 8)
        pltpu.make_async_copy(ex_h.at[pl.ds(eb, _CH)], exb, s1).wait()
        pltpu.make_async_copy(den_h.at[idst.at[i]], dnb, s2).wait()

    def compute(i, sl):
        exb, dnb, atb, _, _ = sl

        def jbody(j, jcarry):
            atb[j, :] = exb[j, :] / (dnb[j, :] + 1e-16)
            return jcarry

        lax.fori_loop(0, _CH, jbody, 0, unroll=8)
        eb = pl.multiple_of(ebase + i * _CH, 8)
        pltpu.sync_copy(atb.at[:, pl.ds(0, _H)], at_h.at[pl.ds(eb, _CH)])

    fire(0, slots[0])

    def gbody(g, carry):
        i0 = 2 * g
        drain(i0, slots[0])
        fire(i0 + 1, slots[1])
        compute(i0, slots[0])
        drain(i0 + 1, slots[1])
        fire(i0 + 2, slots[0])
        compute(i0 + 1, slots[1])
        return carry

    lax.fori_loop(0, (_ITERS - 1) // 2, gbody, 0)
    drain(_ITERS - 1, slots[0])
    compute(_ITERS - 1, slots[0])


def _sc2(dst3, ex_h, dent):
    mesh = plsc.VectorSubcoreMesh(core_axis_name="c", subcore_axis_name="s", num_cores=_NC, num_subcores=_NS)
    f = pl.kernel(
        _sc2_body,
        out_type=[jax.ShapeDtypeStruct((_E, _H), _F32)],
        mesh=mesh,
        scratch_types=[
            pltpu.VMEM((_ITERS, _CH), jnp.int32),
            pltpu.VMEM((_CH, 16), _F32),
            pltpu.VMEM((_CH, 16), _F32),
            pltpu.VMEM((_CH, 16), _F32),
            pltpu.VMEM((_CH, 16), _F32),
            pltpu.VMEM((_CH, 16), _F32),
            pltpu.VMEM((_CH, 16), _F32),
            pltpu.SemaphoreType.DMA,
            pltpu.SemaphoreType.DMA,
            pltpu.SemaphoreType.DMA,
            pltpu.SemaphoreType.DMA,
        ],
        compiler_params=pltpu.CompilerParams(needs_layout_passes=False, use_tc_tiling_on_sc=False),
    )
    return f(dst3, ex_h, dent)[0]


def kernel(x, edge_index, W, att_src, att_dst, bias, mlp_W, mlp_b):
    ei = edge_index.T
    src3 = ei[0].astype(jnp.int32).reshape(_NW, _ITERS, _CH)
    dst3 = ei[1].astype(jnp.int32).reshape(_NW, _ITERS, _CH)

    eye8 = jnp.eye(_C, dtype=_F32)
    a_src = (att_src[:, :, None] * eye8[:, None, :]).reshape(_HC, _H)
    a_dst = (att_dst[:, :, None] * eye8[:, None, :]).reshape(_HC, _H)
    A = jnp.concatenate([a_src, a_dst, a_dst, a_src], axis=1)
    R16 = jnp.concatenate(
        [jnp.repeat(eye8, _C, axis=1), jnp.zeros((_C, _HC), _F32)], axis=0)
    M = jnp.tile(eye8, (_H, 1)) / _H

    xs_tab, t1_tab, t2_tab = _tc_pre(x, W, A)
    ex_h, denp, aggp = _sc1(src3, dst3, t1_tab, t2_tab, xs_tab)
    out, dent = _tc_post(aggp, denp, bias.reshape(1, _C), mlp_W,
                         mlp_b.reshape(1, 16), R16, M)
    attn = _sc2(dst3, ex_h, dent)
    return out, (ei, attn)


# speedup vs baseline: 50.0568x; 1.0459x over previous
import functools
import jax
import jax.numpy as jnp
from jax import lax
from jax.experimental import pallas as pl
from jax.experimental.pallas import tpu as pltpu
from jax.experimental.pallas import tpu_sc as plsc

_N = 10000
_E = 320000
_D = 128
_H = 8
_C = 8
_HC = _H * _C
_NC = 2
_NS = 16
_NW = _NC * _NS
_ET = _E // _NW
_CH = 80
_ITERS = _ET // _CH
_STRIPE = 632
_NP = _STRIPE * _NS
_BLK = 400
_F32 = jnp.float32


def _elu(v):
    return jnp.where(v > 0, v, jnp.exp(v) - 1.0)


def _tc_pre_body(x_ref, w_ref, a_ref, xs_ref, t1_ref, t2_ref):
    xs = jnp.dot(x_ref[...], w_ref[...], preferred_element_type=_F32)
    xs_ref[...] = xs
    t12 = jnp.dot(xs, a_ref[...], preferred_element_type=_F32)
    t1_ref[...] = t12[:, :16]
    t2_ref[...] = t12[:, 16:]


def _tc_pre(x, W, A):
    grid = _N // _BLK
    return pl.pallas_call(
        _tc_pre_body,
        grid=(grid,),
        in_specs=[
            pl.BlockSpec((_BLK, _D), lambda i: (i, 0)),
            pl.BlockSpec((_D, _HC), lambda i: (0, 0)),
            pl.BlockSpec((_HC, 32), lambda i: (0, 0)),
        ],
        out_specs=[
            pl.BlockSpec((_BLK, _HC), lambda i: (i, 0)),
            pl.BlockSpec((_BLK, 16), lambda i: (i, 0)),
            pl.BlockSpec((_BLK, 16), lambda i: (i, 0)),
        ],
        out_shape=[
            jax.ShapeDtypeStruct((_N, _HC), _F32),
            jax.ShapeDtypeStruct((_N, 16), _F32),
            jax.ShapeDtypeStruct((_N, 16), _F32),
        ],
    )(x, W, A)


def _tc_post_body(cp_ref, b_ref, mw_ref, mb_ref,
                  r_ref, m_ref, out_ref, dent_ref):
    comb = cp_ref[0] + cp_ref[1]
    den = comb[:, :16]
    agg = comb[:, 16:]
    dent_ref[...] = 1.0 / (den + 1e-16)
    den64 = jnp.dot(den, r_ref[...], preferred_element_type=_F32) + 1e-16
    agn = agg / den64
    m = jnp.dot(agn, m_ref[...], preferred_element_type=_F32) + b_ref[...]
    m = _elu(m)
    h2 = jnp.dot(m, mw_ref[...], preferred_element_type=_F32) + mb_ref[...]
    out_ref[...] = _elu(h2)


def _tc_post(cp, bias2, mlp_W, mlp_b2, R16, M):
    grid = _N // _BLK
    return pl.pallas_call(
        _tc_post_body,
        grid=(grid,),
        in_specs=[
            pl.BlockSpec((2, _BLK, 80), lambda i: (0, i, 0)),
            pl.BlockSpec((1, _C), lambda i: (0, 0)),
            pl.BlockSpec((_C, 16), lambda i: (0, 0)),
            pl.BlockSpec((1, 16), lambda i: (0, 0)),
            pl.BlockSpec((16, _HC), lambda i: (0, 0)),
            pl.BlockSpec((_HC, _C), lambda i: (0, 0)),
        ],
        out_specs=[
            pl.BlockSpec((_BLK, 16), lambda i: (i, 0)),
            pl.BlockSpec((_BLK, 16), lambda i: (i, 0)),
        ],
        out_shape=[
            jax.ShapeDtypeStruct((_N, 16), _F32),
            jax.ShapeDtypeStruct((_N, 16), _F32),
        ],
    )(cp, bias2, mlp_W, mlp_b2, R16, M)


def _sc1_body(src3_h, dst3_h, t1_h, t2_h, xs_h, ex_h, combp_h,
              isrc, idst, buf_s0, buf_d0, xsb0, buf_s1, buf_d1, xsb1,
              msgb0, msgb1, comb_sh,
              sa0, sb0, sx0, sa1, sb1, sx1):
    c = lax.axis_index("c")
    s = lax.axis_index("s")
    wid = c * _NS + s
    rbase = pl.multiple_of(s * _STRIPE, 8)
    ebase = wid * _ET

    lane = lax.iota(jnp.int32, 16)
    pks = [2 * k + (lane >> 3) for k in range(4)]
    zero16 = jnp.zeros((16,), _F32)

    slots = (
        (buf_s0, buf_d0, xsb0, msgb0, sa0, sb0, sx0),
        (buf_s1, buf_d1, xsb1, msgb1, sa1, sb1, sx1),
    )

    def zbody(j, carry):
        for k in range(5):
            msgb0[j, pl.ds(16 * k, 16)] = zero16
        return carry

    lax.fori_loop(0, _CH, zbody, 0)
    for off in range(0, _STRIPE, _CH):
        n = min(_CH, _STRIPE - off)
        ro = pl.multiple_of(rbase + off, 8)
        pltpu.sync_copy(msgb0.at[pl.ds(0, n)], comb_sh.at[pl.ds(ro, n)])
    plsc.subcore_barrier()

    pltpu.sync_copy(src3_h.at[wid], isrc)
    pltpu.sync_copy(dst3_h.at[wid], idst)

    def fire(i, sl):
        b_s, b_d, b_x, _, s1, s2, s3 = sl
        pltpu.async_copy(t1_h.at[isrc.at[i]], b_s, s1)
        pltpu.async_copy(t2_h.at[idst.at[i]], b_d, s2)
        pltpu.async_copy(xs_h.at[isrc.at[i]], b_x, s3)

    def drain(i, sl):
        b_s, b_d, b_x, _, s1, s2, s3 = sl
        pltpu.make_async_copy(t1_h.at[isrc.at[i]], b_s, s1).wait()
        pltpu.make_async_copy(t2_h.at[idst.at[i]], b_d, s2).wait()
        pltpu.make_async_copy(xs_h.at[isrc.at[i]], b_x, s3).wait()

    def compute(i, sl):
        b_s, b_d, b_x, msgb, _, _, _ = sl

        def jbody(j, jcarry):
            al = b_s[j, :] + b_d[j, :]
            al = jnp.where(al > 0, al, al * 0.2)
            ev = jnp.exp(al)
            msgb[j, pl.ds(0, 16)] = ev
            jv = jnp.full((16,), j, jnp.int32)
            for k in range(4):
                xq = b_x[j, pl.ds(16 * k, 16)]
                mlt = plsc.load_gather(msgb, [jv, pks[k]])
                msgb[j, pl.ds(16 + 16 * k, 16)] = xq * mlt
            return jcarry

        lax.fori_loop(0, _CH, jbody, 0, unroll=4)
        eb = pl.multiple_of(ebase + i * _CH, 8)
        pltpu.sync_copy(msgb.at[:, pl.ds(0, 16)], ex_h.at[pl.ds(eb, _CH)])
        pltpu.sync_copy(msgb, comb_sh.at[idst.at[i]], add=True)

    fire(0, slots[0])

    def gbody(g, carry):
        i0 = 2 * g
        drain(i0, slots[0])
        fire(i0 + 1, slots[1])
        compute(i0, slots[0])
        drain(i0 + 1, slots[1])
        fire(i0 + 2, slots[0])
        compute(i0 + 1, slots[1])
        return carry

    lax.fori_loop(0, (_ITERS - 1) // 2, gbody, 0)
    drain(_ITERS - 1, slots[0])
    compute(_ITERS - 1, slots[0])
    plsc.subcore_barrier()

    for off in range(0, _STRIPE, _CH):
        n = min(_CH, _STRIPE - off)
        ro = pl.multiple_of(rbase + off, 8)
        pltpu.sync_copy(comb_sh.at[pl.ds(ro, n)],
                        combp_h.at[c].at[pl.ds(ro, n)])


def _sc1(src3, dst3, t1_tab, t2_tab, xs_tab):
    mesh = plsc.VectorSubcoreMesh(core_axis_name="c", subcore_axis_name="s", num_cores=_NC, num_subcores=_NS)
    f = pl.kernel(
        _sc1_body,
        out_type=[
            jax.ShapeDtypeStruct((_E, 16), _F32),
            jax.ShapeDtypeStruct((_NC, _NP, 80), _F32),
        ],
        mesh=mesh,
        scratch_types=[
            pltpu.VMEM((_ITERS, _CH), jnp.int32),
            pltpu.VMEM((_ITERS, _CH), jnp.int32),
            pltpu.VMEM((_CH, 16), _F32),
            pltpu.VMEM((_CH, 16), _F32),
            pltpu.VMEM((_CH, _HC), _F32),
            pltpu.VMEM((_CH, 16), _F32),
            pltpu.VMEM((_CH, 16), _F32),
            pltpu.VMEM((_CH, _HC), _F32),
            pltpu.VMEM((_CH, 80), _F32),
            pltpu.VMEM((_CH, 80), _F32),
            pltpu.VMEM_SHARED((_NP, 80), _F32),
            pltpu.SemaphoreType.DMA,
            pltpu.SemaphoreType.DMA,
            pltpu.SemaphoreType.DMA,
            pltpu.SemaphoreType.DMA,
            pltpu.SemaphoreType.DMA,
            pltpu.SemaphoreType.DMA,
        ],
        compiler_params=pltpu.CompilerParams(needs_layout_passes=False, use_tc_tiling_on_sc=False),
    )
    return f(src3, dst3, t1_tab, t2_tab, xs_tab)


def _sc2_body(dst3_h, ex_h, den_h, at_h, idst,
              exb0, dnb0, atb0, exb1, dnb1, atb1, se0, sd0, se1, sd1):
    c = lax.axis_index("c")
    s = lax.axis_index("s")
    wid = c * _NS + s
    ebase = wid * _ET

    pltpu.sync_copy(dst3_h.at[wid], idst)

    slots = ((exb0, dnb0, atb0, se0, sd0), (exb1, dnb1, atb1, se1, sd1))

    def fire(i, sl):
        exb, dnb, _, s1, s2 = sl
        eb = pl.multiple_of(ebase + i * _CH, 8)
        pltpu.async_copy(ex_h.at[pl.ds(eb, _CH)], exb, s1)
        pltpu.async_copy(den_h.at[idst.at[i]], dnb, s2)

    def drain(i, sl):
        exb, dnb, _, s1, s2 = sl
        eb = pl.multiple_of(ebase + i * _CH, 8)
        pltpu.make_async_copy(ex_h.at[pl.ds(eb, _CH)], exb, s1).wait()
        pltpu.make_async_copy(den_h.at[idst.at[i]], dnb, s2).wait()

    def compute(i, sl):
        exb, dnb, atb, _, _ = sl

        def jbody(j, jcarry):
            atb[j, :] = exb[j, :] * dnb[j, :]
            return jcarry

        lax.fori_loop(0, _CH, jbody, 0, unroll=8)
        eb = pl.multiple_of(ebase + i * _CH, 8)
        pltpu.sync_copy(atb.at[:, pl.ds(0, _H)], at_h.at[pl.ds(eb, _CH)])

    fire(0, slots[0])

    def gbody(g, carry):
        i0 = 2 * g
        drain(i0, slots[0])
        fire(i0 + 1, slots[1])
        compute(i0, slots[0])
        drain(i0 + 1, slots[1])
        fire(i0 + 2, slots[0])
        compute(i0 + 1, slots[1])
        return carry

    lax.fori_loop(0, (_ITERS - 1) // 2, gbody, 0)
    drain(_ITERS - 1, slots[0])
    compute(_ITERS - 1, slots[0])


def _sc2(dst3, ex_h, dent):
    mesh = plsc.VectorSubcoreMesh(core_axis_name="c", subcore_axis_name="s", num_cores=_NC, num_subcores=_NS)
    f = pl.kernel(
        _sc2_body,
        out_type=[jax.ShapeDtypeStruct((_E, _H), _F32)],
        mesh=mesh,
        scratch_types=[
            pltpu.VMEM((_ITERS, _CH), jnp.int32),
            pltpu.VMEM((_CH, 16), _F32),
            pltpu.VMEM((_CH, 16), _F32),
            pltpu.VMEM((_CH, 16), _F32),
            pltpu.VMEM((_CH, 16), _F32),
            pltpu.VMEM((_CH, 16), _F32),
            pltpu.VMEM((_CH, 16), _F32),
            pltpu.SemaphoreType.DMA,
            pltpu.SemaphoreType.DMA,
            pltpu.SemaphoreType.DMA,
            pltpu.SemaphoreType.DMA,
        ],
        compiler_params=pltpu.CompilerParams(needs_layout_passes=False, use_tc_tiling_on_sc=False),
    )
    return f(dst3, ex_h, dent)[0]


def kernel(x, edge_index, W, att_src, att_dst, bias, mlp_W, mlp_b):
    ei = edge_index.T
    src3 = ei[0].astype(jnp.int32).reshape(_NW, _ITERS, _CH)
    dst3 = ei[1].astype(jnp.int32).reshape(_NW, _ITERS, _CH)

    eye8 = jnp.eye(_C, dtype=_F32)
    a_src = (att_src[:, :, None] * eye8[:, None, :]).reshape(_HC, _H)
    a_dst = (att_dst[:, :, None] * eye8[:, None, :]).reshape(_HC, _H)
    A = jnp.concatenate([a_src, a_dst, a_dst, a_src], axis=1)
    R16 = jnp.concatenate(
        [jnp.repeat(eye8, _C, axis=1), jnp.zeros((_C, _HC), _F32)], axis=0)
    M = jnp.tile(eye8, (_H, 1)) / _H

    xs_tab, t1_tab, t2_tab = _tc_pre(x, W, A)
    ex_h, combp = _sc1(src3, dst3, t1_tab, t2_tab, xs_tab)
    out, dent = _tc_post(combp, bias.reshape(1, _C), mlp_W,
                         mlp_b.reshape(1, 16), R16, M)
    attn = _sc2(dst3, ex_h, dent)
    return out, (ei, attn)


# speedup vs baseline: 51.8916x; 1.0367x over previous
import functools
import jax
import jax.numpy as jnp
from jax import lax
from jax.experimental import pallas as pl
from jax.experimental.pallas import tpu as pltpu
from jax.experimental.pallas import tpu_sc as plsc

_N = 10000
_E = 320000
_D = 128
_H = 8
_C = 8
_HC = _H * _C
_NC = 2
_NS = 16
_NW = _NC * _NS
_ET = _E // _NW
_CH = 80
_ITERS = _ET // _CH
_STRIPE = 632
_NP = _STRIPE * _NS
_BLK = 400
_F32 = jnp.float32


def _elu(v):
    return jnp.where(v > 0, v, jnp.exp(v) - 1.0)


def _tc_pre_body(x_ref, w_ref, a_ref, ts_ref, t2_ref):
    xs = jnp.dot(x_ref[...], w_ref[...], preferred_element_type=_F32)
    t12 = jnp.dot(xs, a_ref[...], preferred_element_type=_F32)
    ts_ref[...] = jnp.concatenate([t12[:, :16], xs], axis=1)
    t2_ref[...] = t12[:, 16:]


def _tc_pre(x, W, A):
    grid = _N // _BLK
    return pl.pallas_call(
        _tc_pre_body,
        grid=(grid,),
        in_specs=[
            pl.BlockSpec((_BLK, _D), lambda i: (i, 0)),
            pl.BlockSpec((_D, _HC), lambda i: (0, 0)),
            pl.BlockSpec((_HC, 32), lambda i: (0, 0)),
        ],
        out_specs=[
            pl.BlockSpec((_BLK, 80), lambda i: (i, 0)),
            pl.BlockSpec((_BLK, 16), lambda i: (i, 0)),
        ],
        out_shape=[
            jax.ShapeDtypeStruct((_N, 80), _F32),
            jax.ShapeDtypeStruct((_N, 16), _F32),
        ],
    )(x, W, A)


def _tc_post_body(dp_ref, ap_ref, b_ref, mw_ref, mb_ref,
                  r_ref, m_ref, out_ref, dent_ref):
    den = dp_ref[0] + dp_ref[1]
    agg = ap_ref[0] + ap_ref[1]
    dent_ref[...] = 1.0 / (den + 1e-16)
    den64 = jnp.dot(den, r_ref[...], preferred_element_type=_F32) + 1e-16
    agn = agg / den64
    m = jnp.dot(agn, m_ref[...], preferred_element_type=_F32) + b_ref[...]
    m = _elu(m)
    h2 = jnp.dot(m, mw_ref[...], preferred_element_type=_F32) + mb_ref[...]
    out_ref[...] = _elu(h2)


def _tc_post(dp, ap, bias2, mlp_W, mlp_b2, R16, M):
    grid = _N // _BLK
    return pl.pallas_call(
        _tc_post_body,
        grid=(grid,),
        in_specs=[
            pl.BlockSpec((2, _BLK, 16), lambda i: (0, i, 0)),
            pl.BlockSpec((2, _BLK, _HC), lambda i: (0, i, 0)),
            pl.BlockSpec((1, _C), lambda i: (0, 0)),
            pl.BlockSpec((_C, 16), lambda i: (0, 0)),
            pl.BlockSpec((1, 16), lambda i: (0, 0)),
            pl.BlockSpec((16, _HC), lambda i: (0, 0)),
            pl.BlockSpec((_HC, _C), lambda i: (0, 0)),
        ],
        out_specs=[
            pl.BlockSpec((_BLK, 16), lambda i: (i, 0)),
            pl.BlockSpec((_BLK, 16), lambda i: (i, 0)),
        ],
        out_shape=[
            jax.ShapeDtypeStruct((_N, 16), _F32),
            jax.ShapeDtypeStruct((_N, 16), _F32),
        ],
    )(dp, ap, bias2, mlp_W, mlp_b2, R16, M)


def _sc1_body(src3_h, dst3_h, ts_h, t2_h, ex_h, denp_h, aggp_h,
              isrc, idst, buf_s0, buf_d0, buf_s1, buf_d1,
              exb0, msgb0, exb1, msgb1, den_sh, agg_sh,
              sa0, sb0, sa1, sb1):
    c = lax.axis_index("c")
    s = lax.axis_index("s")
    wid = c * _NS + s
    rbase = pl.multiple_of(s * _STRIPE, 8)
    ebase = wid * _ET

    lane = lax.iota(jnp.int32, 16)
    pks = [2 * k + (lane >> 3) for k in range(4)]
    zero16 = jnp.zeros((16,), _F32)

    slots = (
        (buf_s0, buf_d0, exb0, msgb0, sa0, sb0),
        (buf_s1, buf_d1, exb1, msgb1, sa1, sb1),
    )

    def zbody(j, carry):
        exb0[j, :] = zero16
        for k in range(4):
            msgb0[j, pl.ds(16 * k, 16)] = zero16
        return carry

    lax.fori_loop(0, _CH, zbody, 0)
    for off in range(0, _STRIPE, _CH):
        n = min(_CH, _STRIPE - off)
        ro = pl.multiple_of(rbase + off, 8)
        pltpu.sync_copy(msgb0.at[pl.ds(0, n)], agg_sh.at[pl.ds(ro, n)])
        pltpu.sync_copy(exb0.at[pl.ds(0, n)], den_sh.at[pl.ds(ro, n)])
    plsc.subcore_barrier()

    pltpu.sync_copy(src3_h.at[wid], isrc)
    pltpu.sync_copy(dst3_h.at[wid], idst)

    def fire(i, sl):
        b_s, b_d, _, _, s1, s2 = sl
        pltpu.async_copy(ts_h.at[isrc.at[i]], b_s, s1)
        pltpu.async_copy(t2_h.at[idst.at[i]], b_d, s2)

    def drain(i, sl):
        b_s, b_d, _, _, s1, s2 = sl
        pltpu.make_async_copy(ts_h.at[isrc.at[i]], b_s, s1).wait()
        pltpu.make_async_copy(t2_h.at[idst.at[i]], b_d, s2).wait()

    def compute(i, sl):
        b_s, b_d, exb, msgb, _, _ = sl

        def jbody(j, jcarry):
            al = b_s[j, pl.ds(0, 16)] + b_d[j, :]
            al = jnp.where(al > 0, al, al * 0.2)
            ev = jnp.exp(al)
            exb[j, :] = ev
            jv = jnp.full((16,), j, jnp.int32)
            for k in range(4):
                xq = b_s[j, pl.ds(16 + 16 * k, 16)]
                mlt = plsc.load_gather(exb, [jv, pks[k]])
                msgb[j, pl.ds(16 * k, 16)] = xq * mlt
            return jcarry

        lax.fori_loop(0, _CH, jbody, 0, unroll=4)
        eb = pl.multiple_of(ebase + i * _CH, 8)
        pltpu.sync_copy(exb, ex_h.at[pl.ds(eb, _CH)])
        pltpu.sync_copy(exb, den_sh.at[idst.at[i]], add=True)
        pltpu.sync_copy(msgb, agg_sh.at[idst.at[i]], add=True)

    fire(0, slots[0])

    def gbody(g, carry):
        i0 = 2 * g
        drain(i0, slots[0])
        fire(i0 + 1, slots[1])
        compute(i0, slots[0])
        drain(i0 + 1, slots[1])
        fire(i0 + 2, slots[0])
        compute(i0 + 1, slots[1])
        return carry

    lax.fori_loop(0, (_ITERS - 1) // 2, gbody, 0)
    drain(_ITERS - 1, slots[0])
    compute(_ITERS - 1, slots[0])
    plsc.subcore_barrier()

    for off in range(0, _STRIPE, _CH):
        n = min(_CH, _STRIPE - off)
        ro = pl.multiple_of(rbase + off, 8)
        pltpu.sync_copy(den_sh.at[pl.ds(ro, n)],
                        denp_h.at[c].at[pl.ds(ro, n)])
        pltpu.sync_copy(agg_sh.at[pl.ds(ro, n)],
                        aggp_h.at[c].at[pl.ds(ro, n)])


def _sc1(src3, dst3, ts_tab, t2_tab):
    mesh = plsc.VectorSubcoreMesh(core_axis_name="c", subcore_axis_name="s", num_cores=_NC, num_subcores=_NS)
    f = pl.kernel(
        _sc1_body,
        out_type=[
            jax.ShapeDtypeStruct((_E, 16), _F32),
            jax.ShapeDtypeStruct((_NC, _NP, 16), _F32),
            jax.ShapeDtypeStruct((_NC, _NP, _HC), _F32),
        ],
        mesh=mesh,
        scratch_types=[
            pltpu.VMEM((_ITERS, _CH), jnp.int32),
            pltpu.VMEM((_ITERS, _CH), jnp.int32),
            pltpu.VMEM((_CH, 80), _F32),
            pltpu.VMEM((_CH, 16), _F32),
            pltpu.VMEM((_CH, 80), _F32),
            pltpu.VMEM((_CH, 16), _F32),
            pltpu.VMEM((_CH, 16), _F32),
            pltpu.VMEM((_CH, _HC), _F32),
            pltpu.VMEM((_CH, 16), _F32),
            pltpu.VMEM((_CH, _HC), _F32),
            pltpu.VMEM_SHARED((_NP, 16), _F32),
            pltpu.VMEM_SHARED((_NP, _HC), _F32),
            pltpu.SemaphoreType.DMA,
            pltpu.SemaphoreType.DMA,
            pltpu.SemaphoreType.DMA,
            pltpu.SemaphoreType.DMA,
        ],
        compiler_params=pltpu.CompilerParams(needs_layout_passes=False, use_tc_tiling_on_sc=False),
    )
    return f(src3, dst3, ts_tab, t2_tab)


def _sc2_body(dst3_h, ex_h, den_h, at_h, idst,
              exb0, dnb0, atb0, exb1, dnb1, atb1, se0, sd0, se1, sd1):
    c = lax.axis_index("c")
    s = lax.axis_index("s")
    wid = c * _NS + s
    ebase = wid * _ET

    pltpu.sync_copy(dst3_h.at[wid], idst)

    slots = ((exb0, dnb0, atb0, se0, sd0), (exb1, dnb1, atb1, se1, sd1))

    def fire(i, sl):
        exb, dnb, _, s1, s2 = sl
        eb = pl.multiple_of(ebase + i * _CH, 8)
        pltpu.async_copy(ex_h.at[pl.ds(eb, _CH)], exb, s1)
        pltpu.async_copy(den_h.at[idst.at[i]], dnb, s2)

    def drain(i, sl):
        exb, dnb, _, s1, s2 = sl
        eb = pl.multiple_of(ebase + i * _CH, 8)
        pltpu.make_async_copy(ex_h.at[pl.ds(eb, _CH)], exb, s1).wait()
        pltpu.make_async_copy(den_h.at[idst.at[i]], dnb, s2).wait()

    def compute(i, sl):
        exb, dnb, atb, _, _ = sl

        def jbody(j, jcarry):
            atb[j, :] = exb[j, :] * dnb[j, :]
            return jcarry

        lax.fori_loop(0, _CH, jbody, 0, unroll=8)
        eb = pl.multiple_of(ebase + i * _CH, 8)
        pltpu.sync_copy(atb.at[:, pl.ds(0, _H)], at_h.at[pl.ds(eb, _CH)])

    fire(0, slots[0])

    def gbody(g, carry):
        i0 = 2 * g
        drain(i0, slots[0])
        fire(i0 + 1, slots[1])
        compute(i0, slots[0])
        drain(i0 + 1, slots[1])
        fire(i0 + 2, slots[0])
        compute(i0 + 1, slots[1])
        return carry

    lax.fori_loop(0, (_ITERS - 1) // 2, gbody, 0)
    drain(_ITERS - 1, slots[0])
    compute(_ITERS - 1, slots[0])


def _sc2(dst3, ex_h, dent):
    mesh = plsc.VectorSubcoreMesh(core_axis_name="c", subcore_axis_name="s", num_cores=_NC, num_subcores=_NS)
    f = pl.kernel(
        _sc2_body,
        out_type=[jax.ShapeDtypeStruct((_E, _H), _F32)],
        mesh=mesh,
        scratch_types=[
            pltpu.VMEM((_ITERS, _CH), jnp.int32),
            pltpu.VMEM((_CH, 16), _F32),
            pltpu.VMEM((_CH, 16), _F32),
            pltpu.VMEM((_CH, 16), _F32),
            pltpu.VMEM((_CH, 16), _F32),
            pltpu.VMEM((_CH, 16), _F32),
            pltpu.VMEM((_CH, 16), _F32),
            pltpu.SemaphoreType.DMA,
            pltpu.SemaphoreType.DMA,
            pltpu.SemaphoreType.DMA,
            pltpu.SemaphoreType.DMA,
        ],
        compiler_params=pltpu.CompilerParams(needs_layout_passes=False, use_tc_tiling_on_sc=False),
    )
    return f(dst3, ex_h, dent)[0]


def kernel(x, edge_index, W, att_src, att_dst, bias, mlp_W, mlp_b):
    ei = edge_index.T
    src3 = ei[0].astype(jnp.int32).reshape(_NW, _ITERS, _CH)
    dst3 = ei[1].astype(jnp.int32).reshape(_NW, _ITERS, _CH)

    eye8 = jnp.eye(_C, dtype=_F32)
    a_src = (att_src[:, :, None] * eye8[:, None, :]).reshape(_HC, _H)
    a_dst = (att_dst[:, :, None] * eye8[:, None, :]).reshape(_HC, _H)
    A = jnp.concatenate([a_src, a_dst, a_dst, a_src], axis=1)
    R16 = jnp.concatenate(
        [jnp.repeat(eye8, _C, axis=1), jnp.zeros((_C, _HC), _F32)], axis=0)
    M = jnp.tile(eye8, (_H, 1)) / _H

    ts_tab, t2_tab = _tc_pre(x, W, A)
    ex_h, denp, aggp = _sc1(src3, dst3, ts_tab, t2_tab)
    out, dent = _tc_post(denp, aggp, bias.reshape(1, _C), mlp_W,
                         mlp_b.reshape(1, 16), R16, M)
    attn = _sc2(dst3, ex_h, dent)
    return out, (ei, attn)


# speedup vs baseline: 57.4721x; 1.1075x over previous
import functools
import numpy as np
import jax
import jax.numpy as jnp
from jax import lax
from jax.experimental import pallas as pl
from jax.experimental.pallas import tpu as pltpu
from jax.experimental.pallas import tpu_sc as plsc

_N = 10000
_E = 320000
_D = 128
_H = 8
_C = 8
_HC = _H * _C
_NC = 2
_NS = 16
_NW = _NC * _NS
_ET = _E // _NW
_CH = 80
_ITERS = _ET // _CH
_STRIPE = 632
_NP = _STRIPE * _NS
_BLK = 400
_F32 = jnp.float32


def _elu(v):
    return jnp.where(v > 0, v, jnp.exp(v) - 1.0)


def _tc_pre_body(x_ref, w_ref, a_ref, t1_ref, t2_ref, xsb_ref):
    xs = jnp.dot(x_ref[...], w_ref[...], preferred_element_type=_F32)
    t12 = jnp.dot(xs, a_ref[...], preferred_element_type=_F32)
    t1_ref[...] = t12[:, :16]
    t2_ref[...] = t12[:, 16:]
    xsb_ref[...] = xs.astype(jnp.bfloat16)


def _tc_pre(x, W, A):
    grid = _N // _BLK
    return pl.pallas_call(
        _tc_pre_body,
        grid=(grid,),
        in_specs=[
            pl.BlockSpec((_BLK, _D), lambda i: (i, 0)),
            pl.BlockSpec((_D, _HC), lambda i: (0, 0)),
            pl.BlockSpec((_HC, 32), lambda i: (0, 0)),
        ],
        out_specs=[
            pl.BlockSpec((_BLK, 16), lambda i: (i, 0)),
            pl.BlockSpec((_BLK, 16), lambda i: (i, 0)),
            pl.BlockSpec((_BLK, _HC), lambda i: (i, 0)),
        ],
        out_shape=[
            jax.ShapeDtypeStruct((_N, 16), _F32),
            jax.ShapeDtypeStruct((_N, 16), _F32),
            jax.ShapeDtypeStruct((_N, _HC), jnp.bfloat16),
        ],
    )(x, W, A)


def _tc_post_body(dp_ref, ap_ref, b_ref, mw_ref, mb_ref,
                  r_ref, m_ref, out_ref, dent_ref):
    den = dp_ref[0] + dp_ref[1]
    agg = ap_ref[0] + ap_ref[1]
    dent_ref[...] = 1.0 / (den + 1e-16)
    den64 = jnp.dot(den, r_ref[...], preferred_element_type=_F32) + 1e-16
    agn = agg / den64
    m = jnp.dot(agn, m_ref[...], preferred_element_type=_F32) + b_ref[...]
    m = _elu(m)
    h2 = jnp.dot(m, mw_ref[...], preferred_element_type=_F32) + mb_ref[...]
    out_ref[...] = _elu(h2)


def _tc_post(dp, ap, bias2, mlp_W, mlp_b2, R16, M):
    grid = _N // _BLK
    return pl.pallas_call(
        _tc_post_body,
        grid=(grid,),
        in_specs=[
            pl.BlockSpec((2, _BLK, 16), lambda i: (0, i, 0)),
            pl.BlockSpec((2, _BLK, _HC), lambda i: (0, i, 0)),
            pl.BlockSpec((1, _C), lambda i: (0, 0)),
            pl.BlockSpec((_C, 16), lambda i: (0, 0)),
            pl.BlockSpec((1, 16), lambda i: (0, 0)),
            pl.BlockSpec((16, _HC), lambda i: (0, 0)),
            pl.BlockSpec((_HC, _C), lambda i: (0, 0)),
        ],
        out_specs=[
            pl.BlockSpec((_BLK, 16), lambda i: (i, 0)),
            pl.BlockSpec((_BLK, 16), lambda i: (i, 0)),
        ],
        out_shape=[
            jax.ShapeDtypeStruct((_N, 16), _F32),
            jax.ShapeDtypeStruct((_N, 16), _F32),
        ],
    )(dp, ap, bias2, mlp_W, mlp_b2, R16, M)


def _sc1_body(src3_h, dst3_h, t1_h, t2_h, xsb_h, ex_h, denp_h, aggp_h,
              isrc, idst, buf_s0, buf_d0, bx0, buf_s1, buf_d1, bx1,
              exb0, msgb0, exb1, msgb1, den_sh, agg_sh,
              sa0, sb0, sx0, sa1, sb1, sx1):
    c = lax.axis_index("c")
    s = lax.axis_index("s")
    wid = c * _NS + s
    rbase = pl.multiple_of(s * _STRIPE, 8)
    ebase = wid * _ET

    lane = lax.iota(jnp.int32, 16)
    pks = [4 * k2 + (lane >> 2) for k2 in range(2)]
    zero16 = jnp.zeros((16,), _F32)

    slots = (
        (buf_s0, buf_d0, bx0, exb0, msgb0, sa0, sb0, sx0),
        (buf_s1, buf_d1, bx1, exb1, msgb1, sa1, sb1, sx1),
    )

    def zbody(j, carry):
        exb0[j, :] = zero16
        for k in range(4):
            msgb0[j, pl.ds(16 * k, 16)] = zero16
        return carry

    lax.fori_loop(0, _CH, zbody, 0)
    for off in range(0, _STRIPE, _CH):
        n = min(_CH, _STRIPE - off)
        ro = pl.multiple_of(rbase + off, 8)
        pltpu.sync_copy(msgb0.at[pl.ds(0, n)], agg_sh.at[pl.ds(ro, n)])
        pltpu.sync_copy(exb0.at[pl.ds(0, n)], den_sh.at[pl.ds(ro, n)])
    plsc.subcore_barrier()

    pltpu.sync_copy(src3_h.at[wid], isrc)
    pltpu.sync_copy(dst3_h.at[wid], idst)

    def fire(i, sl):
        b_s, b_d, b_x, _, _, s1, s2, s3 = sl
        pltpu.async_copy(t1_h.at[isrc.at[i]], b_s, s1)
        pltpu.async_copy(t2_h.at[idst.at[i]], b_d, s2)
        pltpu.async_copy(xsb_h.at[isrc.at[i]], b_x, s3)

    def drain(i, sl):
        b_s, b_d, b_x, _, _, s1, s2, s3 = sl
        pltpu.make_async_copy(t1_h.at[isrc.at[i]], b_s, s1).wait()
        pltpu.make_async_copy(t2_h.at[idst.at[i]], b_d, s2).wait()
        pltpu.make_async_copy(xsb_h.at[isrc.at[i]], b_x, s3).wait()

    def compute(i, sl):
        b_s, b_d, b_x, exb, msgb, _, _, _ = sl

        def jbody(j, jcarry):
            al = b_s[j, :] + b_d[j, :]
            al = jnp.where(al > 0, al, al * 0.2)
            ev = jnp.exp(al)
            exb[j, :] = ev
            jv = jnp.full((16,), j, jnp.int32)
            for k2 in range(2):
                xq = b_x[j, pl.ds(32 * k2, 32)]
                ae, bo = plsc.unpack(xq, format=plsc.PackFormat.INTERLEAVED)
                mlt = plsc.load_gather(exb, [jv, pks[k2]])
                msgb[j, pl.ds(32 * k2, 16)] = ae * mlt
                msgb[j, pl.ds(32 * k2 + 16, 16)] = bo * mlt
            return jcarry

        lax.fori_loop(0, _CH, jbody, 0, unroll=4)
        eb = pl.multiple_of(ebase + i * _CH, 8)
        pltpu.sync_copy(exb, ex_h.at[pl.ds(eb, _CH)])
        pltpu.sync_copy(exb, den_sh.at[idst.at[i]], add=True)
        pltpu.sync_copy(msgb, agg_sh.at[idst.at[i]], add=True)

    fire(0, slots[0])

    def gbody(g, carry):
        i0 = 2 * g
        drain(i0, slots[0])
        fire(i0 + 1, slots[1])
        compute(i0, slots[0])
        drain(i0 + 1, slots[1])
        fire(i0 + 2, slots[0])
        compute(i0 + 1, slots[1])
        return carry

    lax.fori_loop(0, (_ITERS - 1) // 2, gbody, 0)
    drain(_ITERS - 1, slots[0])
    compute(_ITERS - 1, slots[0])
    plsc.subcore_barrier()

    for off in range(0, _STRIPE, _CH):
        n = min(_CH, _STRIPE - off)
        ro = pl.multiple_of(rbase + off, 8)
        pltpu.sync_copy(den_sh.at[pl.ds(ro, n)],
                        denp_h.at[c].at[pl.ds(ro, n)])
        pltpu.sync_copy(agg_sh.at[pl.ds(ro, n)],
                        aggp_h.at[c].at[pl.ds(ro, n)])


def _sc1(src3, dst3, t1_tab, t2_tab, xsb_tab):
    mesh = plsc.VectorSubcoreMesh(core_axis_name="c", subcore_axis_name="s", num_cores=_NC, num_subcores=_NS)
    f = pl.kernel(
        _sc1_body,
        out_type=[
            jax.ShapeDtypeStruct((_E, 16), _F32),
            jax.ShapeDtypeStruct((_NC, _NP, 16), _F32),
            jax.ShapeDtypeStruct((_NC, _NP, _HC), _F32),
        ],
        mesh=mesh,
        scratch_types=[
            pltpu.VMEM((_ITERS, _CH), jnp.int32),
            pltpu.VMEM((_ITERS, _CH), jnp.int32),
            pltpu.VMEM((_CH, 16), _F32),
            pltpu.VMEM((_CH, 16), _F32),
            pltpu.VMEM((_CH, _HC), jnp.bfloat16),
            pltpu.VMEM((_CH, 16), _F32),
            pltpu.VMEM((_CH, 16), _F32),
            pltpu.VMEM((_CH, _HC), jnp.bfloat16),
            pltpu.VMEM((_CH, 16), _F32),
            pltpu.VMEM((_CH, _HC), _F32),
            pltpu.VMEM((_CH, 16), _F32),
            pltpu.VMEM((_CH, _HC), _F32),
            pltpu.VMEM_SHARED((_NP, 16), _F32),
            pltpu.VMEM_SHARED((_NP, _HC), _F32),
            pltpu.SemaphoreType.DMA,
            pltpu.SemaphoreType.DMA,
            pltpu.SemaphoreType.DMA,
            pltpu.SemaphoreType.DMA,
            pltpu.SemaphoreType.DMA,
            pltpu.SemaphoreType.DMA,
        ],
        compiler_params=pltpu.CompilerParams(needs_layout_passes=False, use_tc_tiling_on_sc=False),
    )
    return f(src3, dst3, t1_tab, t2_tab, xsb_tab)


def _sc2_body(dst3_h, ex_h, den_h, at_h, idst,
              exb0, dnb0, atb0, exb1, dnb1, atb1, se0, sd0, se1, sd1):
    c = lax.axis_index("c")
    s = lax.axis_index("s")
    wid = c * _NS + s
    ebase = wid * _ET

    pltpu.sync_copy(dst3_h.at[wid], idst)

    slots = ((exb0, dnb0, atb0, se0, sd0), (exb1, dnb1, atb1, se1, sd1))

    def fire(i, sl):
        exb, dnb, _, s1, s2 = sl
        eb = pl.multiple_of(ebase + i * _CH, 8)
        pltpu.async_copy(ex_h.at[pl.ds(eb, _CH)], exb, s1)
        pltpu.async_copy(den_h.at[idst.at[i]], dnb, s2)

    def drain(i, sl):
        exb, dnb, _, s1, s2 = sl
        eb = pl.multiple_of(ebase + i * _CH, 8)
        pltpu.make_async_copy(ex_h.at[pl.ds(eb, _CH)], exb, s1).wait()
        pltpu.make_async_copy(den_h.at[idst.at[i]], dnb, s2).wait()

    def compute(i, sl):
        exb, dnb, atb, _, _ = sl

        def jbody(j, jcarry):
            atb[j, :] = exb[j, :] * dnb[j, :]
            return jcarry

        lax.fori_loop(0, _CH, jbody, 0, unroll=8)
        eb = pl.multiple_of(ebase + i * _CH, 8)
        pltpu.sync_copy(atb.at[:, pl.ds(0, _H)], at_h.at[pl.ds(eb, _CH)])

    fire(0, slots[0])

    def gbody(g, carry):
        i0 = 2 * g
        drain(i0, slots[0])
        fire(i0 + 1, slots[1])
        compute(i0, slots[0])
        drain(i0 + 1, slots[1])
        fire(i0 + 2, slots[0])
        compute(i0 + 1, slots[1])
        return carry

    lax.fori_loop(0, (_ITERS - 1) // 2, gbody, 0)
    drain(_ITERS - 1, slots[0])
    compute(_ITERS - 1, slots[0])


def _sc2(dst3, ex_h, dent):
    mesh = plsc.VectorSubcoreMesh(core_axis_name="c", subcore_axis_name="s", num_cores=_NC, num_subcores=_NS)
    f = pl.kernel(
        _sc2_body,
        out_type=[jax.ShapeDtypeStruct((_E, _H), _F32)],
        mesh=mesh,
        scratch_types=[
            pltpu.VMEM((_ITERS, _CH), jnp.int32),
            pltpu.VMEM((_CH, 16), _F32),
            pltpu.VMEM((_CH, 16), _F32),
            pltpu.VMEM((_CH, 16), _F32),
            pltpu.VMEM((_CH, 16), _F32),
            pltpu.VMEM((_CH, 16), _F32),
            pltpu.VMEM((_CH, 16), _F32),
            pltpu.SemaphoreType.DMA,
            pltpu.SemaphoreType.DMA,
            pltpu.SemaphoreType.DMA,
            pltpu.SemaphoreType.DMA,
        ],
        compiler_params=pltpu.CompilerParams(needs_layout_passes=False, use_tc_tiling_on_sc=False),
    )
    return f(dst3, ex_h, dent)[0]


def kernel(x, edge_index, W, att_src, att_dst, bias, mlp_W, mlp_b):
    ei = edge_index.T
    src3 = ei[0].astype(jnp.int32).reshape(_NW, _ITERS, _CH)
    dst3 = ei[1].astype(jnp.int32).reshape(_NW, _ITERS, _CH)

    eye8 = jnp.eye(_C, dtype=_F32)
    a_src = (att_src[:, :, None] * eye8[:, None, :]).reshape(_HC, _H)
    a_dst = (att_dst[:, :, None] * eye8[:, None, :]).reshape(_HC, _H)
    A = jnp.concatenate([a_src, a_dst, a_dst, a_src], axis=1)
    ecols = np.empty(_HC, np.int64)
    for k2 in range(2):
        for r in range(32):
            ecols[32 * k2 + r] = 32 * k2 + (2 * r if r < 16 else
                                            2 * (r - 16) + 1)
    h_of = ecols // _C
    ch_of = ecols % _C
    r16p = np.zeros((16, _HC), np.float32)
    r16p[h_of, np.arange(_HC)] = 1.0
    mp = np.zeros((_HC, _C), np.float32)
    mp[np.arange(_HC), ch_of] = 1.0 / _H
    R16 = jnp.asarray(r16p)
    M = jnp.asarray(mp)

    t1_tab, t2_tab, xsb_tab = _tc_pre(x, W, A)
    ex_h, denp, aggp = _sc1(src3, dst3, t1_tab, t2_tab, xsb_tab)
    out, dent = _tc_post(denp, aggp, bias.reshape(1, _C), mlp_W,
                         mlp_b.reshape(1, 16), R16, M)
    attn = _sc2(dst3, ex_h, dent)
    return out, (ei, attn)


# speedup vs baseline: 58.3122x; 1.0146x over previous
import functools
import numpy as np
import jax
import jax.numpy as jnp
from jax import lax
from jax.experimental import pallas as pl
from jax.experimental.pallas import tpu as pltpu
from jax.experimental.pallas import tpu_sc as plsc

_N = 10000
_E = 320000
_D = 128
_H = 8
_C = 8
_HC = _H * _C
_NC = 2
_NS = 16
_NW = _NC * _NS
_ET = _E // _NW
_CH = 80
_ITERS = _ET // _CH
_STRIPE = 632
_NP = _STRIPE * _NS
_BLK = 400
_F32 = jnp.float32


def _elu(v):
    return jnp.where(v > 0, v, jnp.exp(v) - 1.0)


def _tc_pre_body(x_ref, w_ref, a_ref, t1_ref, t2_ref, xsb_ref):
    xs = jnp.dot(x_ref[...], w_ref[...], preferred_element_type=_F32)
    t12 = jnp.dot(xs, a_ref[...], preferred_element_type=_F32)
    t1_ref[...] = t12[:, :16]
    t2_ref[...] = t12[:, 16:]
    xsb_ref[...] = xs.astype(jnp.bfloat16)


def _tc_pre(x, W, A):
    grid = _N // _BLK
    return pl.pallas_call(
        _tc_pre_body,
        grid=(grid,),
        in_specs=[
            pl.BlockSpec((_BLK, _D), lambda i: (i, 0)),
            pl.BlockSpec((_D, _HC), lambda i: (0, 0)),
            pl.BlockSpec((_HC, 32), lambda i: (0, 0)),
        ],
        out_specs=[
            pl.BlockSpec((_BLK, 16), lambda i: (i, 0)),
            pl.BlockSpec((_BLK, 16), lambda i: (i, 0)),
            pl.BlockSpec((_BLK, _HC), lambda i: (i, 0)),
        ],
        out_shape=[
            jax.ShapeDtypeStruct((_N, 16), _F32),
            jax.ShapeDtypeStruct((_N, 16), _F32),
            jax.ShapeDtypeStruct((_N, _HC), jnp.bfloat16),
        ],
    )(x, W, A)


def _tc_post_body(dp_ref, ap_ref, b_ref, mw_ref, mb_ref,
                  r_ref, m_ref, out_ref, dent_ref):
    den = dp_ref[0] + dp_ref[1]
    agg = ap_ref[0] + ap_ref[1]
    dent_ref[...] = 1.0 / (den + 1e-16)
    den64 = jnp.dot(den, r_ref[...], preferred_element_type=_F32) + 1e-16
    agn = agg / den64
    m = jnp.dot(agn, m_ref[...], preferred_element_type=_F32) + b_ref[...]
    m = _elu(m)
    h2 = jnp.dot(m, mw_ref[...], preferred_element_type=_F32) + mb_ref[...]
    out_ref[...] = _elu(h2)


def _tc_post(dp, ap, bias2, mlp_W, mlp_b2, R16, M):
    grid = _N // _BLK
    return pl.pallas_call(
        _tc_post_body,
        grid=(grid,),
        in_specs=[
            pl.BlockSpec((2, _BLK, 16), lambda i: (0, i, 0)),
            pl.BlockSpec((2, _BLK, _HC), lambda i: (0, i, 0)),
            pl.BlockSpec((1, _C), lambda i: (0, 0)),
            pl.BlockSpec((_C, 16), lambda i: (0, 0)),
            pl.BlockSpec((1, 16), lambda i: (0, 0)),
            pl.BlockSpec((16, _HC), lambda i: (0, 0)),
            pl.BlockSpec((_HC, _C), lambda i: (0, 0)),
        ],
        out_specs=[
            pl.BlockSpec((_BLK, 16), lambda i: (i, 0)),
            pl.BlockSpec((_BLK, 16), lambda i: (i, 0)),
        ],
        out_shape=[
            jax.ShapeDtypeStruct((_N, 16), _F32),
            jax.ShapeDtypeStruct((_N, 16), _F32),
        ],
    )(dp, ap, bias2, mlp_W, mlp_b2, R16, M)


def _sc1_body(src3_h, dst3_h, t1_h, t2_h, xsb_h, ex_h, denp_h, aggp_h,
              isrc, idst, buf_s0, buf_d0, bx0, buf_s1, buf_d1, bx1,
              exb0, msgb0, exb1, msgb1, den_sh, agg_sh,
              sa0, sb0, sx0, sa1, sb1, sx1):
    c = lax.axis_index("c")
    s = lax.axis_index("s")
    wid = c * _NS + s
    rbase = pl.multiple_of(s * _STRIPE, 8)
    ebase = wid * _ET

    lane = lax.iota(jnp.int32, 16)
    pks = [4 * k2 + (lane >> 2) for k2 in range(2)]
    zero16 = jnp.zeros((16,), _F32)

    slots = (
        (buf_s0, buf_d0, bx0, exb0, msgb0, sa0, sb0, sx0),
        (buf_s1, buf_d1, bx1, exb1, msgb1, sa1, sb1, sx1),
    )

    def zbody(j, carry):
        exb0[j, :] = zero16
        for k in range(4):
            msgb0[j, pl.ds(16 * k, 16)] = zero16
        return carry

    lax.fori_loop(0, _CH, zbody, 0)
    for off in range(0, _STRIPE, _CH):
        n = min(_CH, _STRIPE - off)
        ro = pl.multiple_of(rbase + off, 8)
        pltpu.sync_copy(msgb0.at[pl.ds(0, n)], agg_sh.at[pl.ds(ro, n)])
        pltpu.sync_copy(exb0.at[pl.ds(0, n)], den_sh.at[pl.ds(ro, n)])
    plsc.subcore_barrier()

    pltpu.sync_copy(src3_h.at[wid], isrc)
    pltpu.sync_copy(dst3_h.at[wid], idst)

    def fire(i, sl):
        b_s, b_d, b_x, _, _, s1, s2, s3 = sl
        pltpu.async_copy(t1_h.at[isrc.at[i]], b_s, s1)
        pltpu.async_copy(t2_h.at[idst.at[i]], b_d, s2)
        pltpu.async_copy(xsb_h.at[isrc.at[i]], b_x, s3)

    def drain(i, sl):
        b_s, b_d, b_x, _, _, s1, s2, s3 = sl
        pltpu.make_async_copy(t1_h.at[isrc.at[i]], b_s, s1).wait()
        pltpu.make_async_copy(t2_h.at[idst.at[i]], b_d, s2).wait()
        pltpu.make_async_copy(xsb_h.at[isrc.at[i]], b_x, s3).wait()

    def compute(i, sl):
        b_s, b_d, b_x, exb, msgb, _, _, _ = sl

        def jbody(j, jcarry):
            al = b_s[j, :] + b_d[j, :]
            al = jnp.where(al > 0, al, al * 0.2)
            ev = jnp.exp(al)
            exb[j, :] = ev
            jv = jnp.full((16,), j, jnp.int32)
            for k2 in range(2):
                xq = b_x[j, pl.ds(32 * k2, 32)]
                ae, bo = plsc.unpack(xq, format=plsc.PackFormat.INTERLEAVED)
                mlt = plsc.load_gather(exb, [jv, pks[k2]])
                msgb[j, pl.ds(32 * k2, 16)] = ae * mlt
                msgb[j, pl.ds(32 * k2 + 16, 16)] = bo * mlt
            return jcarry

        lax.fori_loop(0, _CH, jbody, 0, unroll=8)
        eb = pl.multiple_of(ebase + i * _CH, 8)
        pltpu.sync_copy(exb, ex_h.at[pl.ds(eb, _CH)])
        pltpu.sync_copy(exb, den_sh.at[idst.at[i]], add=True)
        pltpu.sync_copy(msgb, agg_sh.at[idst.at[i]], add=True)

    fire(0, slots[0])

    def gbody(g, carry):
        i0 = 2 * g
        drain(i0, slots[0])
        fire(i0 + 1, slots[1])
        compute(i0, slots[0])
        drain(i0 + 1, slots[1])
        fire(i0 + 2, slots[0])
        compute(i0 + 1, slots[1])
        return carry

    lax.fori_loop(0, (_ITERS - 1) // 2, gbody, 0)
    drain(_ITERS - 1, slots[0])
    compute(_ITERS - 1, slots[0])
    plsc.subcore_barrier()

    for off in range(0, _STRIPE, _CH):
        n = min(_CH, _STRIPE - off)
        ro = pl.multiple_of(rbase + off, 8)
        pltpu.sync_copy(den_sh.at[pl.ds(ro, n)],
                        denp_h.at[c].at[pl.ds(ro, n)])
        pltpu.sync_copy(agg_sh.at[pl.ds(ro, n)],
                        aggp_h.at[c].at[pl.ds(ro, n)])


def _sc1(src3, dst3, t1_tab, t2_tab, xsb_tab):
    mesh = plsc.VectorSubcoreMesh(core_axis_name="c", subcore_axis_name="s", num_cores=_NC, num_subcores=_NS)
    f = pl.kernel(
        _sc1_body,
        out_type=[
            jax.ShapeDtypeStruct((_E, 16), _F32),
            jax.ShapeDtypeStruct((_NC, _NP, 16), _F32),
            jax.ShapeDtypeStruct((_NC, _NP, _HC), _F32),
        ],
        mesh=mesh,
        scratch_types=[
            pltpu.VMEM((_ITERS, _CH), jnp.int32),
            pltpu.VMEM((_ITERS, _CH), jnp.int32),
            pltpu.VMEM((_CH, 16), _F32),
            pltpu.VMEM((_CH, 16), _F32),
            pltpu.VMEM((_CH, _HC), jnp.bfloat16),
            pltpu.VMEM((_CH, 16), _F32),
            pltpu.VMEM((_CH, 16), _F32),
            pltpu.VMEM((_CH, _HC), jnp.bfloat16),
            pltpu.VMEM((_CH, 16), _F32),
            pltpu.VMEM((_CH, _HC), _F32),
            pltpu.VMEM((_CH, 16), _F32),
            pltpu.VMEM((_CH, _HC), _F32),
            pltpu.VMEM_SHARED((_NP, 16), _F32),
            pltpu.VMEM_SHARED((_NP, _HC), _F32),
            pltpu.SemaphoreType.DMA,
            pltpu.SemaphoreType.DMA,
            pltpu.SemaphoreType.DMA,
            pltpu.SemaphoreType.DMA,
            pltpu.SemaphoreType.DMA,
            pltpu.SemaphoreType.DMA,
        ],
        compiler_params=pltpu.CompilerParams(needs_layout_passes=False, use_tc_tiling_on_sc=False),
    )
    return f(src3, dst3, t1_tab, t2_tab, xsb_tab)


def _sc2_body(dst3_h, ex_h, den_h, at_h, idst,
              exb0, dnb0, atb0, exb1, dnb1, atb1, se0, sd0, se1, sd1):
    c = lax.axis_index("c")
    s = lax.axis_index("s")
    wid = c * _NS + s
    ebase = wid * _ET

    pltpu.sync_copy(dst3_h.at[wid], idst)

    slots = ((exb0, dnb0, atb0, se0, sd0), (exb1, dnb1, atb1, se1, sd1))

    def fire(i, sl):
        exb, dnb, _, s1, s2 = sl
        eb = pl.multiple_of(ebase + i * _CH, 8)
        pltpu.async_copy(ex_h.at[pl.ds(eb, _CH)], exb, s1)
        pltpu.async_copy(den_h.at[idst.at[i]], dnb, s2)

    def drain(i, sl):
        exb, dnb, _, s1, s2 = sl
        eb = pl.multiple_of(ebase + i * _CH, 8)
        pltpu.make_async_copy(ex_h.at[pl.ds(eb, _CH)], exb, s1).wait()
        pltpu.make_async_copy(den_h.at[idst.at[i]], dnb, s2).wait()

    def compute(i, sl):
        exb, dnb, atb, _, _ = sl

        def jbody(j, jcarry):
            atb[j, :] = exb[j, :] * dnb[j, :]
            return jcarry

        lax.fori_loop(0, _CH, jbody, 0, unroll=8)
        eb = pl.multiple_of(ebase + i * _CH, 8)
        pltpu.sync_copy(atb.at[:, pl.ds(0, _H)], at_h.at[pl.ds(eb, _CH)])

    fire(0, slots[0])

    def gbody(g, carry):
        i0 = 2 * g
        drain(i0, slots[0])
        fire(i0 + 1, slots[1])
        compute(i0, slots[0])
        drain(i0 + 1, slots[1])
        fire(i0 + 2, slots[0])
        compute(i0 + 1, slots[1])
        return carry

    lax.fori_loop(0, (_ITERS - 1) // 2, gbody, 0)
    drain(_ITERS - 1, slots[0])
    compute(_ITERS - 1, slots[0])


def _sc2(dst3, ex_h, dent):
    mesh = plsc.VectorSubcoreMesh(core_axis_name="c", subcore_axis_name="s", num_cores=_NC, num_subcores=_NS)
    f = pl.kernel(
        _sc2_body,
        out_type=[jax.ShapeDtypeStruct((_E, _H), _F32)],
        mesh=mesh,
        scratch_types=[
            pltpu.VMEM((_ITERS, _CH), jnp.int32),
            pltpu.VMEM((_CH, 16), _F32),
            pltpu.VMEM((_CH, 16), _F32),
            pltpu.VMEM((_CH, 16), _F32),
            pltpu.VMEM((_CH, 16), _F32),
            pltpu.VMEM((_CH, 16), _F32),
            pltpu.VMEM((_CH, 16), _F32),
            pltpu.SemaphoreType.DMA,
            pltpu.SemaphoreType.DMA,
            pltpu.SemaphoreType.DMA,
            pltpu.SemaphoreType.DMA,
        ],
        compiler_params=pltpu.CompilerParams(needs_layout_passes=False, use_tc_tiling_on_sc=False),
    )
    return f(dst3, ex_h, dent)[0]


def kernel(x, edge_index, W, att_src, att_dst, bias, mlp_W, mlp_b):
    ei = edge_index.T
    src3 = ei[0].astype(jnp.int32).reshape(_NW, _ITERS, _CH)
    dst3 = ei[1].astype(jnp.int32).reshape(_NW, _ITERS, _CH)

    eye8 = jnp.eye(_C, dtype=_F32)
    a_src = (att_src[:, :, None] * eye8[:, None, :]).reshape(_HC, _H)
    a_dst = (att_dst[:, :, None] * eye8[:, None, :]).reshape(_HC, _H)
    A = jnp.concatenate([a_src, a_dst, a_dst, a_src], axis=1)
    ecols = np.empty(_HC, np.int64)
    for k2 in range(2):
        for r in range(32):
            ecols[32 * k2 + r] = 32 * k2 + (2 * r if r < 16 else
                                            2 * (r - 16) + 1)
    h_of = ecols // _C
    ch_of = ecols % _C
    r16p = np.zeros((16, _HC), np.float32)
    r16p[h_of, np.arange(_HC)] = 1.0
    mp = np.zeros((_HC, _C), np.float32)
    mp[np.arange(_HC), ch_of] = 1.0 / _H
    R16 = jnp.asarray(r16p)
    M = jnp.asarray(mp)

    t1_tab, t2_tab, xsb_tab = _tc_pre(x, W, A)
    ex_h, denp, aggp = _sc1(src3, dst3, t1_tab, t2_tab, xsb_tab)
    out, dent = _tc_post(denp, aggp, bias.reshape(1, _C), mlp_W,
                         mlp_b.reshape(1, 16), R16, M)
    attn = _sc2(dst3, ex_h, dent)
    return out, (ei, attn)
